# Initial kernel scaffold; baseline (speedup 1.0000x reference)
import jax
import jax.numpy as jnp
from jax.experimental import pallas as pl


def kernel(x, bias_values):
    raise NotImplementedError("write your pallas kernel here")


# speedup vs baseline: 25.1086x; 25.1086x over previous
import functools

import jax
import jax.numpy as jnp
from jax import lax
from jax.experimental import pallas as pl
from jax.experimental.pallas import tpu as pltpu
from jax.experimental.pallas import tpu_sc as plsc

_LANES = 16
_NUM_CORES = 2
_NUM_SUBCORES = 16
_NUM_WORKERS = _NUM_CORES * _NUM_SUBCORES
_CHUNK = 128


@functools.lru_cache(maxsize=None)
def _build_sc_kernel(S: int, num_buckets: int, num_heads: int):
    L = _LANES
    NW = _NUM_WORKERS
    rows_total = num_heads * S
    assert rows_total % NW == 0
    rows_per_worker = rows_total // NW
    assert rows_per_worker % 8 == 0 and S % rows_per_worker == 0
    P = 2 * S
    assert P % _CHUNK == 0
    md = num_buckets // 2
    shift = md - (S - 1)
    n_p8 = 8 * P

    mesh = plsc.VectorSubcoreMesh(core_axis_name="c", subcore_axis_name="s")

    @functools.partial(
        pl.kernel,
        mesh=mesh,
        out_type=jax.ShapeDtypeStruct((rows_total, S), jnp.float32),
        compiler_params=pltpu.CompilerParams(use_tc_tiling_on_sc=False),
        scratch_types=[
            pltpu.VMEM((n_p8,), jnp.int32),
            pltpu.VMEM((n_p8,), jnp.float32),
            pltpu.SemaphoreType.DMA,
        ],
    )
    def sc_kernel(bt_hbm, out_hbm, idx_v, p8_v, sem):
        wid = lax.axis_index("s") * _NUM_CORES + lax.axis_index("c")
        row0 = wid * rows_per_worker
        h = row0 // S
        i0 = row0 - h * S

        iota = lax.iota(jnp.int32, L)
        hbase = h * num_buckets

        def build_idx(t, _):
            base = t * L
            r = base // P
            u = (base - r * P) + iota
            idx_v[pl.ds(base, L)] = hbase + jnp.clip(
                u + (r + shift), 0, num_buckets - 1
            )
            return 0

        lax.fori_loop(0, n_p8 // L, build_idx, 0, unroll=False)

        K = 8
        n_chunks = n_p8 // _CHUNK

        def gather(g, _):
            handles = []
            for b in range(K):
                off = (g * K + b) * _CHUNK
                src = bt_hbm.at[idx_v.at[pl.ds(off, _CHUNK)]]
                handles.append(pltpu.async_copy(src, p8_v.at[pl.ds(off, _CHUNK)], sem))
            for hd in handles:
                hd.wait()
            return 0

        lax.fori_loop(0, n_chunks // K, gather, 0, unroll=False)

        def rows(g, _):
            i_base = i0 + g * K
            handles = []
            for b in range(K):
                i = i_base + b
                start = (S - 1) - i
                r = jnp.bitwise_and(start, 7)
                off = pl.multiple_of(r * P + (start - r), 8)
                src = p8_v.at[pl.ds(off, S)]
                dst = out_hbm.at[h * S + i]
                handles.append(pltpu.async_copy(src, dst, sem))
            for hd in handles:
                hd.wait()
            return 0

        lax.fori_loop(0, rows_per_worker // K, rows, 0, unroll=False)

    return sc_kernel


def kernel(x, bias_values):
    S = x.shape[1]
    num_buckets, num_heads = bias_values.shape
    sc = _build_sc_kernel(S, num_buckets, num_heads)
    bt = bias_values.astype(jnp.float32).T.reshape(-1)
    out = sc(bt)
    return out.reshape(1, num_heads, S, S)


# speedup vs baseline: 25.9506x; 1.0335x over previous
import functools

import jax
import jax.numpy as jnp
from jax import lax
from jax.experimental import pallas as pl
from jax.experimental.pallas import tpu as pltpu
from jax.experimental.pallas import tpu_sc as plsc

_LANES = 16
_NUM_CORES = 2
_NUM_SUBCORES = 16
_NUM_WORKERS = _NUM_CORES * _NUM_SUBCORES
_CHUNK = 128


@functools.lru_cache(maxsize=None)
def _build_sc_kernel(S: int, num_buckets: int, num_heads: int):
    L = _LANES
    NW = _NUM_WORKERS
    rows_total = num_heads * S
    assert rows_total % NW == 0
    rows_per_worker = rows_total // NW
    assert rows_per_worker % 8 == 0 and S % rows_per_worker == 0
    P = 2 * S
    assert P % _CHUNK == 0
    md = num_buckets // 2
    shift = md - (S - 1)
    n_p8 = 8 * P

    mesh = plsc.VectorSubcoreMesh(core_axis_name="c", subcore_axis_name="s")

    @functools.partial(
        pl.kernel,
        mesh=mesh,
        out_type=jax.ShapeDtypeStruct((rows_total, S), jnp.float32),
        compiler_params=pltpu.CompilerParams(use_tc_tiling_on_sc=False),
        scratch_types=[
            pltpu.VMEM((n_p8,), jnp.int32),
            pltpu.VMEM((n_p8,), jnp.float32),
            pltpu.SemaphoreType.DMA,
        ],
    )
    def sc_kernel(bt_hbm, out_hbm, idx_v, p8_v, sem):
        wid = lax.axis_index("s") * _NUM_CORES + lax.axis_index("c")
        row0 = wid * rows_per_worker
        h = row0 // S
        i0 = row0 - h * S

        iota = lax.iota(jnp.int32, L)
        hbase = h * num_buckets

        def build_idx(slot, _):
            base_u = slot * L
            c0 = (base_u + shift) + iota
            for r in range(8):
                idx_v[pl.ds(r * P + base_u, L)] = hbase + jnp.clip(
                    c0 + r, 0, num_buckets - 1
                )
            return 0

        lax.fori_loop(0, P // L, build_idx, 0, unroll=False)

        K = 8
        n_chunks = n_p8 // _CHUNK

        def gather(g, _):
            handles = []
            for b in range(K):
                off = (g * K + b) * _CHUNK
                src = bt_hbm.at[idx_v.at[pl.ds(off, _CHUNK)]]
                handles.append(pltpu.async_copy(src, p8_v.at[pl.ds(off, _CHUNK)], sem))
            for hd in handles:
                hd.wait()
            return 0

        lax.fori_loop(0, n_chunks // K, gather, 0, unroll=False)

        def fire_row(i):
            start = (S - 1) - i
            r = jnp.bitwise_and(start, 7)
            off = pl.multiple_of(r * P + (start - r), 8)
            src = p8_v.at[pl.ds(off, S)]
            dst = out_hbm.at[h * S + i]
            return pltpu.async_copy(src, dst, sem)

        DEPTH = 16
        B = 4
        for b in range(DEPTH):
            fire_row(i0 + b)

        def rows(g, _):
            i_base = i0 + DEPTH + g * B
            handles = [fire_row(i_base + b) for b in range(B)]
            for hd in handles:
                hd.wait()
            return 0

        lax.fori_loop(0, (rows_per_worker - DEPTH) // B, rows, 0, unroll=False)
        for b in range(DEPTH):
            pltpu.make_async_copy(
                out_hbm.at[h * S + i0], p8_v.at[pl.ds(0, S)], sem
            ).wait()

    return sc_kernel


def kernel(x, bias_values):
    S = x.shape[1]
    num_buckets, num_heads = bias_values.shape
    sc = _build_sc_kernel(S, num_buckets, num_heads)
    bt = bias_values.astype(jnp.float32).T.reshape(-1)
    out = sc(bt)
    return out.reshape(1, num_heads, S, S)
